# Initial kernel scaffold; baseline (speedup 1.0000x reference)
import functools

import jax
import jax.numpy as jnp
from jax import lax
from jax.experimental import pallas as pl
from jax.experimental.pallas import tpu as pltpu
from jax.experimental.pallas import tpu_sc as plsc

N = 10000
E = 320000
D = 128
C = 128
DE = 16

NTILES = 32
EPT = E // NTILES
CH = 80
NCHUNK = EPT // CH
WC = 144
STRIPE = N // 16

_HIGH = jax.lax.Precision.HIGHEST


def _dense_body(x_ref, w_ref, asv_ref, adv_ref, h_ref, as_ref, ad_ref):
    h = jnp.dot(x_ref[...], w_ref[...], preferred_element_type=jnp.float32,
                precision=_HIGH)
    h_ref[...] = h
    as_ref[...] = (h * asv_ref[...][None, :]).sum(axis=1)
    ad_ref[...] = (h * adv_ref[...][None, :]).sum(axis=1)


_dense = pl.pallas_call(
    _dense_body,
    out_shape=[
        jax.ShapeDtypeStruct((N, C), jnp.float32),
        jax.ShapeDtypeStruct((N,), jnp.float32),
        jax.ShapeDtypeStruct((N,), jnp.float32),
    ],
)


def _fuse_body(raws_ref, b_ref, w_ref, asv_ref, adv_ref, h_ref, as_ref, ad_ref):
    r = raws_ref[0, :, :C] + raws_ref[1, :, :C]
    dn = raws_ref[0, :, C] + raws_ref[1, :, C]
    hin = jnp.maximum(r / (dn + 1e-16)[:, None] + b_ref[...][None, :], 0.0)
    h = jnp.dot(hin, w_ref[...], preferred_element_type=jnp.float32,
                precision=_HIGH)
    h_ref[...] = h
    as_ref[...] = (h * asv_ref[...][None, :]).sum(axis=1)
    ad_ref[...] = (h * adv_ref[...][None, :]).sum(axis=1)


_fuse = pl.pallas_call(
    _fuse_body,
    out_shape=[
        jax.ShapeDtypeStruct((N, C), jnp.float32),
        jax.ShapeDtypeStruct((N,), jnp.float32),
        jax.ShapeDtypeStruct((N,), jnp.float32),
    ],
)


def _final_body(raws_ref, b_ref, out_ref):
    r = raws_ref[0, :, :C] + raws_ref[1, :, :C]
    dn = raws_ref[0, :, C] + raws_ref[1, :, C]
    out_ref[...] = r / (dn + 1e-16)[:, None] + b_ref[...][None, :]


_final = pl.pallas_call(
    _final_body,
    out_shape=jax.ShapeDtypeStruct((N, C), jnp.float32),
)


def _ae_body(ea_ref, wc_ref, out_ref):
    out_ref[...] = jnp.dot(ea_ref[...], wc_ref[...],
                           preferred_element_type=jnp.float32, precision=_HIGH)


_AE_BLK = 20000
_ae_map = pl.pallas_call(
    _ae_body,
    grid=(E // _AE_BLK,),
    in_specs=[
        pl.BlockSpec((_AE_BLK, DE), lambda i: (i, 0)),
        pl.BlockSpec((DE, 8), lambda i: (0, 0)),
    ],
    out_specs=pl.BlockSpec((_AE_BLK, 8), lambda i: (i, 0)),
    out_shape=jax.ShapeDtypeStruct((E, 8), jnp.float32),
)


_mesh = plsc.VectorSubcoreMesh(core_axis_name="c", subcore_axis_name="s")


@functools.partial(
    pl.kernel,
    out_type=jax.ShapeDtypeStruct((2, N, WC), jnp.float32),
    mesh=_mesh,
    scratch_types=[
        pltpu.VMEM((NCHUNK, CH), jnp.int32),
        pltpu.VMEM((NCHUNK, CH), jnp.int32),
        pltpu.VMEM((NCHUNK, CH), jnp.float32),
        pltpu.VMEM((N,), jnp.float32),
        pltpu.VMEM((N,), jnp.float32),
        pltpu.VMEM((CH, C), jnp.float32),
        pltpu.VMEM((CH, WC), jnp.float32),
        pltpu.VMEM_SHARED((N, WC), jnp.float32),
    ],
)
def _sc_edge(src_hbm, dst_hbm, ae_hbm, as_hbm, ad_hbm, h_hbm, z_hbm,
             out_hbm, srcL, dstL, aeL, asL, adL, A, B, acc):
    cid = lax.axis_index("c")
    sid = lax.axis_index("s")
    wid = cid * 16 + sid

    pltpu.sync_copy(src_hbm.at[wid], srcL)
    pltpu.sync_copy(dst_hbm.at[wid], dstL)
    pltpu.sync_copy(ae_hbm.at[wid], aeL)
    pltpu.sync_copy(as_hbm, asL)
    pltpu.sync_copy(ad_hbm, adL)
    pltpu.sync_copy(z_hbm, acc.at[pl.ds(sid * STRIPE, STRIPE)])
    pltpu.sync_copy(z_hbm.at[pl.ds(0, CH)], B)
    plsc.subcore_barrier()

    @pl.loop(0, NCHUNK)
    def _chunk(ci):
        pltpu.sync_copy(h_hbm.at[srcL.at[ci]], A)
        for k in range(CH // 16):
            s16 = srcL[ci, pl.ds(k * 16, 16)]
            d16 = dstL[ci, pl.ds(k * 16, 16)]
            al = (plsc.load_gather(asL, [s16])
                  + plsc.load_gather(adL, [d16])
                  + aeL[ci, pl.ds(k * 16, 16)])
            al = jnp.where(al >= 0.0, al, 0.2 * al)
            ex = jnp.exp(al)
            rows = k * 16 + lax.iota(jnp.int32, 16)
            cols = jnp.full((16,), C, jnp.int32)
            plsc.store_scatter(B, [rows, cols], ex)
        for e in range(CH):
            sc = B[e, C]
            for j in range(C // 16):
                B[e, pl.ds(j * 16, 16)] = A[e, pl.ds(j * 16, 16)] * sc
        pltpu.sync_copy(B, acc.at[dstL.at[ci]], add=True)

    plsc.subcore_barrier()
    pltpu.sync_copy(acc.at[pl.ds(sid * STRIPE, STRIPE)],
                    out_hbm.at[cid, pl.ds(sid * STRIPE, STRIPE)])


def kernel(x, edge_index, edge_attr, W1, att_src1, att_dst1, We1, att_e1, b1,
           W2, att_src2, att_dst2, We2, att_e2, b2,
           W3, att_src3, att_dst3, We3, att_e3, b3):
    f32 = jnp.float32
    src3 = edge_index[0].reshape(NTILES, NCHUNK, CH)
    dst3 = edge_index[1].reshape(NTILES, NCHUNK, CH)
    zrows = jnp.zeros((STRIPE, WC), f32)

    wcat = jnp.zeros((DE, 8), f32)
    for i, (We, ae) in enumerate(((We1, att_e1), (We2, att_e2), (We3, att_e3))):
        wcat = wcat.at[:, i].set(We @ ae.reshape(C))
    ae8 = _ae_map(edge_attr, wcat)
    ae_l = [ae8[:, i].reshape(NTILES, NCHUNK, CH) for i in range(3)]

    h, a_s, a_d = _dense(x, W1, att_src1.reshape(C), att_dst1.reshape(C))
    raws = _sc_edge(src3, dst3, ae_l[0], a_s, a_d, h, zrows)
    h, a_s, a_d = _fuse(raws, b1, W2, att_src2.reshape(C), att_dst2.reshape(C))
    raws = _sc_edge(src3, dst3, ae_l[1], a_s, a_d, h, zrows)
    h, a_s, a_d = _fuse(raws, b2, W3, att_src3.reshape(C), att_dst3.reshape(C))
    raws = _sc_edge(src3, dst3, ae_l[2], a_s, a_d, h, zrows)
    out = _final(raws, b3)
    return out.reshape(1, N, C)


# speedup vs baseline: 19.9600x; 19.9600x over previous
import functools

import jax
import jax.numpy as jnp
from jax import lax
from jax.experimental import pallas as pl
from jax.experimental.pallas import tpu as pltpu
from jax.experimental.pallas import tpu_sc as plsc

N = 10000
E = 320000
D = 128
C = 128
DE = 16

NTILES = 32
EPT = E // NTILES
CH = 80
NCHUNK = EPT // CH
HALF = 5000
ACCR = 5008
GARB = HALF

_HIGH = jax.lax.Precision.HIGHEST


def _dense_body(x_ref, w_ref, asv_ref, adv_ref, h_ref, as_ref, ad_ref):
    h = jnp.dot(x_ref[...], w_ref[...], preferred_element_type=jnp.float32,
                precision=_HIGH)
    h_ref[...] = h
    as_ref[...] = (h * asv_ref[...][None, :]).sum(axis=1)
    ad_ref[...] = (h * adv_ref[...][None, :]).sum(axis=1)


_dense = pl.pallas_call(
    _dense_body,
    out_shape=[
        jax.ShapeDtypeStruct((N, C), jnp.float32),
        jax.ShapeDtypeStruct((N,), jnp.float32),
        jax.ShapeDtypeStruct((N,), jnp.float32),
    ],
)


def _fuse_body(raws_ref, dens_ref, b_ref, w_ref, asv_ref, adv_ref,
               nm_ref, h_ref, as_ref, ad_ref):
    r = raws_ref[0] + raws_ref[1]
    dn = dens_ref[...].sum(axis=0)
    nm = r / (dn + 1e-16)[:, None] + b_ref[...][None, :]
    nm_ref[...] = nm
    h = jnp.dot(jnp.maximum(nm, 0.0), w_ref[...],
                preferred_element_type=jnp.float32, precision=_HIGH)
    h_ref[...] = h
    as_ref[...] = (h * asv_ref[...][None, :]).sum(axis=1)
    ad_ref[...] = (h * adv_ref[...][None, :]).sum(axis=1)


_fuse = pl.pallas_call(
    _fuse_body,
    out_shape=[
        jax.ShapeDtypeStruct((N, C), jnp.float32),
        jax.ShapeDtypeStruct((N, C), jnp.float32),
        jax.ShapeDtypeStruct((N,), jnp.float32),
        jax.ShapeDtypeStruct((N,), jnp.float32),
    ],
)


def _ae_body(wc_ref, ea_ref, out_ref):
    out_ref[...] = jnp.dot(wc_ref[...], ea_ref[...],
                           preferred_element_type=jnp.float32, precision=_HIGH)


_AE_BLK = 16000
_ae_map = pl.pallas_call(
    _ae_body,
    grid=(E // _AE_BLK,),
    in_specs=[
        pl.BlockSpec((8, DE), lambda i: (0, 0)),
        pl.BlockSpec((DE, _AE_BLK), lambda i: (0, i)),
    ],
    out_specs=pl.BlockSpec((8, _AE_BLK), lambda i: (0, i)),
    out_shape=jax.ShapeDtypeStruct((8, E), jnp.float32),
)


_mesh = plsc.VectorSubcoreMesh(core_axis_name="c", subcore_axis_name="s")


@functools.partial(
    pl.kernel,
    out_type=(
        pltpu.HBM((2, N, C), jnp.float32),
        pltpu.HBM((NTILES, N), jnp.float32),
    ),
    mesh=_mesh,
    compiler_params=pltpu.CompilerParams(needs_layout_passes=False),
    scratch_types=[
        pltpu.VMEM((EPT,), jnp.int32),
        pltpu.VMEM((EPT,), jnp.int32),
        pltpu.VMEM((EPT,), jnp.float32),
        pltpu.VMEM((N,), jnp.float32),
        pltpu.VMEM((N,), jnp.float32),
        pltpu.VMEM((N,), jnp.float32),
        pltpu.VMEM((CH, C), jnp.float32),
        pltpu.VMEM((CH, C), jnp.float32),
        pltpu.VMEM((CH,), jnp.int32),
        pltpu.VMEM((CH,), jnp.int32),
        pltpu.VMEM((CH,), jnp.int32),
        pltpu.VMEM((CH,), jnp.int32),
        pltpu.SemaphoreType.DMA,
        pltpu.SemaphoreType.DMA,
        pltpu.SemaphoreType.DMA,
        pltpu.SemaphoreType.DMA,
        pltpu.VMEM_SHARED((ACCR, C), jnp.float32),
    ],
)
def _sc_edge(src_hbm, dst_hbm, ae_hbm, as_hbm, ad_hbm, h_hbm, z_hbm,
             out_hbm, outd_hbm, srcL, dstL, aeL, asL, adL, denL,
             A0, A1, si0, si1, di0, di1, sg0, sg1, ss0, ss1, acc):
    A = (A0, A1)
    si = (si0, si1)
    di = (di0, di1)
    sg = (sg0, sg1)
    ss = (ss0, ss1)
    cid = lax.axis_index("c")
    sid = lax.axis_index("s")
    wid = cid * 16 + sid

    pltpu.sync_copy(src_hbm.at[wid], srcL)
    pltpu.sync_copy(dst_hbm.at[wid], dstL)
    pltpu.sync_copy(ae_hbm.at[wid], aeL)
    pltpu.sync_copy(as_hbm, asL)
    pltpu.sync_copy(ad_hbm, adL)

    zero16 = jnp.zeros((16,), jnp.float32)

    @pl.loop(0, N, step=16)
    def _zero_den(i):
        denL[pl.ds(i, 16)] = zero16

    for p in range(2):
        base = p * HALF
        @pl.when(sid < 15)
        def _zero_main():
            pltpu.sync_copy(z_hbm.at[pl.ds(0, 312)],
                            acc.at[pl.ds(sid * 312, 312)])

        @pl.when(sid == 15)
        def _zero_tail():
            pltpu.sync_copy(z_hbm, acc.at[pl.ds(15 * 312, 328)])

        plsc.subcore_barrier()

        def _fill(b, cc):
            for k in range(CH // 16):
                s16 = srcL[pl.ds(cc * CH + k * 16, 16)]
                d16 = dstL[pl.ds(cc * CH + k * 16, 16)]
                si[b][pl.ds(k * 16, 16)] = s16
                inr = (d16 >= base) & (d16 < base + HALF)
                di[b][pl.ds(k * 16, 16)] = jnp.where(inr, d16 - base, GARB)

        def _wait_gather(b):
            pltpu.make_async_copy(h_hbm.at[si[b]], A[b], sg[b]).wait()

        def _wait_scatter(b):
            pltpu.make_async_copy(A[b], acc.at[di[b]], ss[b]).wait()

        def _compute(b, cc):
            for k in range(CH // 16):
                s16 = srcL[pl.ds(cc * CH + k * 16, 16)]
                d16 = dstL[pl.ds(cc * CH + k * 16, 16)]
                al = (plsc.load_gather(asL, [s16])
                      + plsc.load_gather(adL, [d16])
                      + aeL[pl.ds(cc * CH + k * 16, 16)])
                al = jnp.where(al >= 0.0, al, 0.2 * al)
                ex = jnp.exp(al)
                if p == 0:
                    plsc.addupdate_scatter(denL, [d16], ex)
                for e in range(16):
                    sc = ex[e]
                    row = k * 16 + e
                    for j in range(C // 16):
                        A[b][row, pl.ds(j * 16, 16)] = (
                            A[b][row, pl.ds(j * 16, 16)] * sc)

        for k in range(CH // 16):
            di[1][pl.ds(k * 16, 16)] = jnp.full((16,), GARB, jnp.int32)
        pltpu.async_copy(A[1], acc.at[di[1]], ss[1], add=True)
        _fill(0, 0)
        pltpu.async_copy(h_hbm.at[si[0]], A[0], sg[0])

        @pl.loop(0, NCHUNK - 1, step=2)
        def _chunk(ci):
            for b in range(2):
                _wait_scatter(1 - b)
                _fill(1 - b, ci + b + 1)
                pltpu.async_copy(h_hbm.at[si[1 - b]], A[1 - b], sg[1 - b])
                _wait_gather(b)
                _compute(b, ci + b)
                pltpu.async_copy(A[b], acc.at[di[b]], ss[b], add=True)

        _wait_gather(0)
        _compute(0, NCHUNK - 1)
        pltpu.sync_copy(A[0], acc.at[di[0]], add=True)
        _wait_scatter(1)

        if p == 0:
            pltpu.sync_copy(denL, outd_hbm.at[wid])
        plsc.subcore_barrier()
        @pl.when(sid < 15)
        def _wb_main():
            pltpu.sync_copy(acc.at[pl.ds(sid * 312, 312)],
                            out_hbm.at[cid, pl.ds(base + sid * 312, 312)])

        @pl.when(sid == 15)
        def _wb_tail():
            pltpu.sync_copy(acc.at[pl.ds(15 * 312, 320)],
                            out_hbm.at[cid, pl.ds(base + 15 * 312, 320)])

        plsc.subcore_barrier()


def kernel(x, edge_index, edge_attr, W1, att_src1, att_dst1, We1, att_e1, b1,
           W2, att_src2, att_dst2, We2, att_e2, b2,
           W3, att_src3, att_dst3, We3, att_e3, b3):
    f32 = jnp.float32
    src3 = edge_index[0].reshape(NTILES, EPT)
    dst3 = edge_index[1].reshape(NTILES, EPT)
    zrows = jnp.zeros((328, C), f32)

    wcat = jnp.zeros((8, DE), f32)
    for i, (We, ae) in enumerate(((We1, att_e1), (We2, att_e2), (We3, att_e3))):
        wcat = wcat.at[i, :].set(We @ ae.reshape(C))
    ae8 = _ae_map(wcat, edge_attr.T)
    ae_l = [ae8[i].reshape(NTILES, EPT) for i in range(3)]

    ae_stack = jnp.stack(ae_l)
    b_stack = jnp.stack([b1, b2, b3])
    w_stack = jnp.stack([W2, W3, jnp.zeros((C, C), f32)])
    asv_stack = jnp.stack([att_src2.reshape(C), att_src3.reshape(C),
                           jnp.zeros((C,), f32)])
    adv_stack = jnp.stack([att_dst2.reshape(C), att_dst3.reshape(C),
                           jnp.zeros((C,), f32)])

    h, a_s, a_d = _dense(x, W1, att_src1.reshape(C), att_dst1.reshape(C))

    def _layer(carry, xs):
        h, a_s, a_d = carry
        ae_i, b_i, w_i, asv_i, adv_i = xs
        raws, dens = _sc_edge(src3, dst3, ae_i, a_s, a_d, h, zrows)
        nm, h2, as2, ad2 = _fuse(raws, dens, b_i, w_i, asv_i, adv_i)
        return (h2, as2, ad2), nm

    _, nms = jax.lax.scan(
        _layer, (h, a_s, a_d),
        (ae_stack, b_stack, w_stack, asv_stack, adv_stack))
    return nms[2].reshape(1, N, C)
